# baseline (speedup 1.0000x reference)
import functools

import jax
import jax.numpy as jnp
from jax import lax
from jax.experimental import pallas as pl
from jax.experimental.pallas import tpu as pltpu
from jax.experimental.pallas import tpu_sc as plsc

N_NODES = 10000
DEG = 32
D = 128

BN = 400
N_TC = 6800
S_SC = N_NODES - N_TC
NBLK_TC = N_TC // BN
NBLK_SC = S_SC // BN

NW = 32
NPW = S_SC // NW
CH = 2
NBUF = 2
NCH = NPW // CH


def _fused_low_body(x_ref, neigh_ref, wx_ref, bx_ref, wn_ref, bn_ref, out_ref):
    nb = neigh_ref[...].reshape(BN, DEG, D)
    agg = jnp.sum(nb, axis=1) * (1.0 / DEG)
    h_x = lax.dot_general(
        x_ref[...], wx_ref[...], (((1,), (1,)), ((), ())),
        preferred_element_type=jnp.float32)
    h_n = lax.dot_general(
        agg, wn_ref[...], (((1,), (1,)), ((), ())),
        preferred_element_type=jnp.float32)
    out_ref[:, :D] = jnp.maximum(h_x + bx_ref[...], 0.0)
    out_ref[:, D:] = jnp.maximum(h_n + bn_ref[...], 0.0)


def _high_body(prev_ref, x_ref, agg_ref, wx_ref, bx_ref, wn_ref, bn_ref,
               out_ref):
    del prev_ref
    h_x = lax.dot_general(
        x_ref[...], wx_ref[...], (((1,), (1,)), ((), ())),
        preferred_element_type=jnp.float32)
    h_n = lax.dot_general(
        agg_ref[...], wn_ref[...], (((1,), (1,)), ((), ())),
        preferred_element_type=jnp.float32)
    out_ref[:, :D] = jnp.maximum(h_x + bx_ref[...], 0.0)
    out_ref[:, D:] = jnp.maximum(h_n + bn_ref[...], 0.0)


_SC_MESH = plsc.VectorSubcoreMesh(core_axis_name="c", subcore_axis_name="s")


@functools.partial(
    pl.kernel,
    out_type=jax.ShapeDtypeStruct((S_SC, D), jnp.float32),
    mesh=_SC_MESH,
    scratch_types=[
        pltpu.VMEM((CH * DEG, D), jnp.float32),
        pltpu.VMEM((CH * DEG, D), jnp.float32),
        pltpu.VMEM((CH, D), jnp.float32),
        pltpu.VMEM((CH, D), jnp.float32),
        pltpu.SemaphoreType.DMA,
        pltpu.SemaphoreType.DMA,
    ],
)
def _sc_agg(neigh_hbm, agg_hbm, buf0, buf1, ob0, ob1, sem0, sem1):
    wid = lax.axis_index("s") * 2 + lax.axis_index("c")
    base_node = wid * NPW

    def src_slice(c):
        row0 = (N_TC + base_node + c * CH) * DEG
        return neigh_hbm.at[pl.ds(row0, CH * DEG)]

    pltpu.async_copy(src_slice(0), buf0, sem0)
    pltpu.async_copy(src_slice(1), buf1, sem1)

    def body(i, carry):
        for b in range(NBUF):
            buf, sem, ob = ((buf0, sem0, ob0), (buf1, sem1, ob1))[b]
            c = i * NBUF + b
            pltpu.make_async_copy(src_slice(c), buf, sem).wait()
            for n in range(CH):
                for g in range(D // 16):
                    sl = pl.ds(g * 16, 16)
                    vals = [buf[n * DEG + r, sl] for r in range(DEG)]
                    while len(vals) > 1:
                        vals = [vals[2 * j] + vals[2 * j + 1]
                                for j in range(len(vals) // 2)]
                    ob[n, sl] = vals[0] * (1.0 / DEG)

            @pl.when(c + NBUF < NCH)
            def _():
                pltpu.async_copy(src_slice(c + NBUF), buf, sem)

            pltpu.sync_copy(ob, agg_hbm.at[pl.ds(base_node + c * CH, CH)])
        return carry

    lax.fori_loop(0, NCH // NBUF, body, 0)


@jax.jit
def _hybrid(x, neigh, W_x, b_x, W_n, b_n):
    agg_hi = _sc_agg(neigh)

    out_low = pl.pallas_call(
        _fused_low_body,
        grid=(NBLK_TC,),
        in_specs=[
            pl.BlockSpec((BN, D), lambda i: (i, 0)),
            pl.BlockSpec((BN * DEG, D), lambda i: (i, 0)),
            pl.BlockSpec((D, D), lambda i: (0, 0)),
            pl.BlockSpec((1, D), lambda i: (0, 0)),
            pl.BlockSpec((D, D), lambda i: (0, 0)),
            pl.BlockSpec((1, D), lambda i: (0, 0)),
        ],
        out_specs=pl.BlockSpec((BN, 2 * D), lambda i: (i, 0)),
        out_shape=jax.ShapeDtypeStruct((N_NODES, 2 * D), jnp.float32),
    )(x, neigh, W_x, b_x, W_n, b_n)

    out = pl.pallas_call(
        _high_body,
        grid=(NBLK_SC,),
        in_specs=[
            pl.BlockSpec(memory_space=pl.ANY),
            pl.BlockSpec((BN, D), lambda i: (NBLK_TC + i, 0)),
            pl.BlockSpec((BN, D), lambda i: (i, 0)),
            pl.BlockSpec((D, D), lambda i: (0, 0)),
            pl.BlockSpec((1, D), lambda i: (0, 0)),
            pl.BlockSpec((D, D), lambda i: (0, 0)),
            pl.BlockSpec((1, D), lambda i: (0, 0)),
        ],
        out_specs=pl.BlockSpec((BN, 2 * D), lambda i: (NBLK_TC + i, 0)),
        out_shape=jax.ShapeDtypeStruct((N_NODES, 2 * D), jnp.float32),
        input_output_aliases={0: 0},
    )(out_low, x, agg_hi, W_x, b_x, W_n, b_n)
    return out


def kernel(x, neigh, W_x, b_x, W_n, b_n):
    return _hybrid(x, neigh, W_x.reshape(D, D), b_x.reshape(1, D),
                   W_n.reshape(D, D), b_n.reshape(1, D))


# speedup vs baseline: 1.9897x; 1.9897x over previous
import functools

import jax
import jax.numpy as jnp
from jax import lax
from jax.experimental import pallas as pl

N_NODES = 10000
DEG = 32
D = 128

BN = 400
NBLK = N_NODES // BN


def _fused_body(x_ref, neigh_ref, wx_ref, bx_ref, wn_ref, bn_ref, out_ref):
    nb = neigh_ref[...].reshape(BN, DEG, D)
    agg = jnp.sum(nb, axis=1) * (1.0 / DEG)
    h_x = lax.dot_general(
        x_ref[...], wx_ref[...], (((1,), (1,)), ((), ())),
        preferred_element_type=jnp.float32)
    h_n = lax.dot_general(
        agg, wn_ref[...], (((1,), (1,)), ((), ())),
        preferred_element_type=jnp.float32)
    out_ref[:, :D] = jnp.maximum(h_x + bx_ref[...], 0.0)
    out_ref[:, D:] = jnp.maximum(h_n + bn_ref[...], 0.0)


@jax.jit
def _fused(x, neigh, W_x, b_x, W_n, b_n):
    return pl.pallas_call(
        _fused_body,
        grid=(NBLK,),
        in_specs=[
            pl.BlockSpec((BN, D), lambda i: (i, 0)),
            pl.BlockSpec((BN * DEG, D), lambda i: (i, 0)),
            pl.BlockSpec((D, D), lambda i: (0, 0)),
            pl.BlockSpec((1, D), lambda i: (0, 0)),
            pl.BlockSpec((D, D), lambda i: (0, 0)),
            pl.BlockSpec((1, D), lambda i: (0, 0)),
        ],
        out_specs=pl.BlockSpec((BN, 2 * D), lambda i: (i, 0)),
        out_shape=jax.ShapeDtypeStruct((N_NODES, 2 * D), jnp.float32),
    )(x, neigh, W_x, b_x, W_n, b_n)


def kernel(x, neigh, W_x, b_x, W_n, b_n):
    return _fused(x, neigh, W_x.reshape(D, D), b_x.reshape(1, D),
                  W_n.reshape(D, D), b_n.reshape(1, D))
